# baseline (speedup 1.0000x reference)
import functools
import math

import jax
import jax.numpy as jnp
from jax import lax
from jax.experimental import pallas as pl
from jax.experimental.pallas import tpu as pltpu
from jax.experimental.pallas import tpu_sc as plsc

_VOCAB = 1000000
_DIM = 64
_B = 16384
_L = 200
_N = _B * _L
_NC = 2
_NS = 16
_NW = _NC * _NS
_PER_W = _N // _NW
_CHUNK = 200
_STEPS = _PER_W // _CHUNK
_SCALE = math.sqrt(_DIM)

_mesh = plsc.VectorSubcoreMesh(core_axis_name="c", subcore_axis_name="s")


@functools.partial(
    pl.kernel,
    out_type=jax.ShapeDtypeStruct((_N, _DIM), jnp.float32),
    mesh=_mesh,
    scratch_types=[
        pltpu.VMEM((_CHUNK,), jnp.int32),
        pltpu.VMEM((_CHUNK,), jnp.int32),
        pltpu.VMEM((_CHUNK, 2 * _DIM), jnp.float32),
        pltpu.VMEM((_CHUNK, 2 * _DIM), jnp.float32),
        pltpu.VMEM((_CHUNK, _DIM), jnp.float32),
        pltpu.VMEM((_CHUNK, _DIM), jnp.float32),
        pltpu.SemaphoreType.DMA,
        pltpu.SemaphoreType.DMA,
        pltpu.SemaphoreType.DMA,
        pltpu.SemaphoreType.DMA,
    ],
    compiler_params=pltpu.CompilerParams(
        needs_layout_passes=False, use_tc_tiling_on_sc=True),
)
def _embed_gather(table_hbm, idx_hbm, out_hbm,
                  idx0, idx1, rows0, rows1, ob0, ob1, g0, g1, s0, s1):
    wid = lax.axis_index("s") * _NC + lax.axis_index("c")
    base = wid * _PER_W
    idx_v = (idx0, idx1)
    rows_v = (rows0, rows1)
    out_v = (ob0, ob1)
    gsem = (g0, g1)
    ssem = (s0, s1)

    def stage(i, b):
        pltpu.sync_copy(idx_hbm.at[pl.ds(base + i * _CHUNK, _CHUNK)], idx_v[b])
        pltpu.async_copy(table_hbm.at[idx_v[b]], rows_v[b], gsem[b])

    def scale_rows(b):
        rv = rows_v[b]
        ob = out_v[b]

        def scale_row(r, carry):
            for c in range(_DIM // 16):
                sl = pl.ds(c * 16, 16)
                ob[r, sl] = rv[r, sl] * _SCALE
            return carry

        lax.fori_loop(0, _CHUNK, scale_row, 0, unroll=4)

    stage(0, 0)

    def outer(g, carry):
        for b in range(2):
            i = 2 * g + b
            nb = 1 - b
            pltpu.make_async_copy(table_hbm.at[idx_v[b]], rows_v[b], gsem[b]).wait()
            if b == 0:
                @pl.when(g > 0)
                def _wait_prev_store():
                    pltpu.make_async_copy(
                        out_v[nb], out_hbm.at[pl.ds(base, _CHUNK)], ssem[nb]).wait()
                stage(i + 1, nb)
            else:
                @pl.when(g < _STEPS // 2 - 1)
                def _prefetch():
                    pltpu.make_async_copy(
                        out_v[nb], out_hbm.at[pl.ds(base, _CHUNK)], ssem[nb]).wait()
                    stage(i + 1, nb)
            scale_rows(b)
            pltpu.async_copy(
                out_v[b], out_hbm.at[pl.ds(base + i * _CHUNK, _CHUNK)], ssem[b])
        return carry

    lax.fori_loop(0, _STEPS // 2, outer, 0)
    pltpu.make_async_copy(ob0, out_hbm.at[pl.ds(base, _CHUNK)], s0).wait()
    pltpu.make_async_copy(ob1, out_hbm.at[pl.ds(base, _CHUNK)], s1).wait()


def kernel(tokens, table):
    flat = tokens.reshape(_N)
    table_pad = jnp.pad(table, ((0, 0), (0, _DIM)))
    out = _embed_gather(table_pad, flat)
    return out.reshape(_B, _L, _DIM)


# speedup vs baseline: 1.0879x; 1.0879x over previous
import functools
import math

import jax
import jax.numpy as jnp
from jax import lax
from jax.experimental import pallas as pl
from jax.experimental.pallas import tpu as pltpu
from jax.experimental.pallas import tpu_sc as plsc

_VOCAB = 1000000
_DIM = 64
_B = 16384
_L = 200
_N = _B * _L
_NC = 2
_NS = 16
_NW = _NC * _NS
_PER_W = _N // _NW
_CHUNK = 200
_STEPS = _PER_W // _CHUNK
_SCALE = math.sqrt(_DIM)

_mesh = plsc.VectorSubcoreMesh(core_axis_name="c", subcore_axis_name="s")


@functools.partial(
    pl.kernel,
    out_type=jax.ShapeDtypeStruct((_N, _DIM), jnp.float32),
    mesh=_mesh,
    scratch_types=[
        pltpu.VMEM((_PER_W // 4,), jnp.int32),
        pltpu.VMEM((_CHUNK, 2 * _DIM), jnp.float32),
        pltpu.VMEM((_CHUNK, 2 * _DIM), jnp.float32),
        pltpu.VMEM((_CHUNK, _DIM), jnp.float32),
        pltpu.VMEM((_CHUNK, _DIM), jnp.float32),
        pltpu.SemaphoreType.DMA,
        pltpu.SemaphoreType.DMA,
        pltpu.SemaphoreType.DMA,
        pltpu.SemaphoreType.DMA,
    ],
    compiler_params=pltpu.CompilerParams(
        needs_layout_passes=False, use_tc_tiling_on_sc=True),
)
def _embed_gather(table_hbm, idx_hbm, out_hbm,
                  idx_big, rows0, rows1, ob0, ob1, g0, g1, s0, s1):
    wid = lax.axis_index("s") * _NC + lax.axis_index("c")
    base = wid * _PER_W
    rows_v = (rows0, rows1)
    out_v = (ob0, ob1)
    gsem = (g0, g1)
    ssem = (s0, s1)
    _QSTEPS = _STEPS // 4

    def scale_rows(b):
        rv = rows_v[b]
        ob = out_v[b]

        def scale_row(r, carry):
            for c in range(_DIM // 16):
                sl = pl.ds(c * 16, 16)
                ob[r, sl] = rv[r, sl] * _SCALE
            return carry

        lax.fori_loop(0, _CHUNK, scale_row, 0, unroll=4)

    for q in range(4):
        if q:
            pltpu.make_async_copy(ob0, out_hbm.at[pl.ds(base, _CHUNK)], s0).wait()
            pltpu.make_async_copy(ob1, out_hbm.at[pl.ds(base, _CHUNK)], s1).wait()
        qoff = base + q * _QSTEPS * _CHUNK
        pltpu.sync_copy(idx_hbm.at[pl.ds(qoff, _QSTEPS * _CHUNK)], idx_big)

        def stage(i, b):
            pltpu.async_copy(
                table_hbm.at[idx_big.at[pl.ds(i * _CHUNK, _CHUNK)]],
                rows_v[b], gsem[b])

        stage(0, 0)

        def outer(g, carry):
            for b in range(2):
                i = 2 * g + b
                nb = 1 - b
                pltpu.make_async_copy(
                    table_hbm.at[idx_big.at[pl.ds(i * _CHUNK, _CHUNK)]],
                    rows_v[b], gsem[b]).wait()
                if b == 0:
                    @pl.when(g > 0)
                    def _wait_prev_store():
                        pltpu.make_async_copy(
                            out_v[nb], out_hbm.at[pl.ds(base, _CHUNK)],
                            ssem[nb]).wait()
                    stage(i + 1, nb)
                else:
                    @pl.when(g < _QSTEPS // 2 - 1)
                    def _prefetch():
                        pltpu.make_async_copy(
                            out_v[nb], out_hbm.at[pl.ds(base, _CHUNK)],
                            ssem[nb]).wait()
                        stage(i + 1, nb)
                scale_rows(b)
                pltpu.async_copy(
                    out_v[b], out_hbm.at[pl.ds(qoff + i * _CHUNK, _CHUNK)],
                    ssem[b])
            return carry

        lax.fori_loop(0, _QSTEPS // 2, outer, 0)

    pltpu.make_async_copy(ob0, out_hbm.at[pl.ds(base, _CHUNK)], s0).wait()
    pltpu.make_async_copy(ob1, out_hbm.at[pl.ds(base, _CHUNK)], s1).wait()


def kernel(tokens, table):
    flat = tokens.reshape(_N)
    table_pad = jnp.pad(table, ((0, 0), (0, _DIM)))
    out = _embed_gather(table_pad, flat)
    return out.reshape(_B, _L, _DIM)


# speedup vs baseline: 1.1613x; 1.0674x over previous
import functools
import math

import jax
import jax.numpy as jnp
from jax import lax
from jax.experimental import pallas as pl
from jax.experimental.pallas import tpu as pltpu
from jax.experimental.pallas import tpu_sc as plsc

_VOCAB = 1000000
_DIM = 64
_B = 16384
_L = 200
_N = _B * _L
_NC = 2
_NS = 16
_NW = _NC * _NS
_CHUNK = 800
_SCALE = math.sqrt(_DIM)

_mesh = plsc.VectorSubcoreMesh(core_axis_name="c", subcore_axis_name="s")


def _make_gather(n):
    per_w = n // _NW
    steps = per_w // _CHUNK

    @functools.partial(
        pl.kernel,
        out_type=jax.ShapeDtypeStruct((n, _DIM), jnp.float32),
        mesh=_mesh,
        scratch_types=[
            pltpu.VMEM((_CHUNK,), jnp.int32),
            pltpu.VMEM((_CHUNK,), jnp.int32),
            pltpu.VMEM((_CHUNK, _DIM), jnp.float32),
            pltpu.VMEM((_CHUNK, _DIM), jnp.float32),
            pltpu.SemaphoreType.DMA,
            pltpu.SemaphoreType.DMA,
            pltpu.SemaphoreType.DMA,
            pltpu.SemaphoreType.DMA,
        ],
        compiler_params=pltpu.CompilerParams(use_tc_tiling_on_sc=False),
    )
    def _embed_gather(table_hbm, idx_hbm, out_hbm,
                      idx0, idx1, rows0, rows1, g0, g1, s0, s1):
        wid = lax.axis_index("s") * _NC + lax.axis_index("c")
        base = wid * per_w
        idx_v = (idx0, idx1)
        rows_v = (rows0, rows1)
        gsem = (g0, g1)
        ssem = (s0, s1)

        def stage(i, b):
            pltpu.sync_copy(idx_hbm.at[pl.ds(base + i * _CHUNK, _CHUNK)], idx_v[b])
            pltpu.async_copy(table_hbm.at[idx_v[b]], rows_v[b], gsem[b])

        def scale_rows(rv):
            def scale_row(r, carry):
                for c in range(_DIM // 16):
                    sl = pl.ds(c * 16, 16)
                    rv[r, sl] = rv[r, sl] * _SCALE
                return carry
            lax.fori_loop(0, _CHUNK, scale_row, 0, unroll=4)

        stage(0, 0)

        def outer(g, carry):
            for b in range(2):
                i = 2 * g + b
                nb = 1 - b
                pltpu.make_async_copy(
                    table_hbm.at[idx_v[b]], rows_v[b], gsem[b]).wait()
                if b == 0:
                    @pl.when(g > 0)
                    def _wait_prev_store():
                        pltpu.make_async_copy(
                            rows_v[nb], out_hbm.at[pl.ds(base, _CHUNK)],
                            ssem[nb]).wait()
                    stage(i + 1, nb)
                else:
                    @pl.when(g < steps // 2 - 1)
                    def _prefetch():
                        pltpu.make_async_copy(
                            rows_v[nb], out_hbm.at[pl.ds(base, _CHUNK)],
                            ssem[nb]).wait()
                        stage(i + 1, nb)
                scale_rows(rows_v[b])
                pltpu.async_copy(
                    rows_v[b], out_hbm.at[pl.ds(base + i * _CHUNK, _CHUNK)],
                    ssem[b])
            return carry

        lax.fori_loop(0, steps // 2, outer, 0)
        pltpu.make_async_copy(rows0, out_hbm.at[pl.ds(base, _CHUNK)], s0).wait()
        pltpu.make_async_copy(rows1, out_hbm.at[pl.ds(base, _CHUNK)], s1).wait()

    return _embed_gather


_half_gather = _make_gather(_N // 2)


def kernel(tokens, table):
    flat = tokens.reshape(_N)
    a = _half_gather(table, flat[:_N // 2]).reshape(_B // 2, _L, _DIM)
    b = _half_gather(table, flat[_N // 2:]).reshape(_B // 2, _L, _DIM)
    return jnp.concatenate([a, b], axis=0)
